# Initial kernel scaffold; baseline (speedup 1.0000x reference)
import jax
import jax.numpy as jnp
from jax.experimental import pallas as pl


def kernel(x, edge_index, W_l, b_l, W_r):
    raise NotImplementedError("write your pallas kernel here")


# speedup vs baseline: 5.4318x; 5.4318x over previous
import functools

import jax
import jax.numpy as jnp
from jax import lax
from jax.experimental import pallas as pl
from jax.experimental.pallas import tpu as pltpu
from jax.experimental.pallas import tpu_sc as plsc

N = 10000
E = 160000
D = 256
H = 256

NC = 2
NS = 16
DH = D // 2
CH = 80
EPT = E // NS
NCH = EPT // CH
NROWCH = E // CH
N_PAD = 10240
GCH = 25
NG = NCH // GCH
IDXR = 32
SLAB = NROWCH
RPT = N_PAD // NS
WCH = 64
NW = RPT // WCH


def _sc_scatter_kernel(x2_hbm, src_hbm, dst_hbm, agg_hbm, cnt_hbm,
                       src_v, dst_v, rows_v, ones_v, zero_v, czero_v,
                       stage_v, cstage_v, acc_sh, cnt_sh):
    cid = lax.axis_index("c")
    sid = lax.axis_index("s")

    zf = jnp.zeros((16,), jnp.float32)
    for r in range(16):
        for c in range(DH // 16):
            zero_v[r, pl.ds(c * 16, 16)] = zf
    for c in range(RPT // 16):
        czero_v[pl.ds(c * 16, 16)] = zf
    of = jnp.ones((16,), jnp.float32)
    for c in range(CH // 16):
        ones_v[pl.ds(c * 16, 16)] = of

    for w in range(RPT // 16):
        pltpu.sync_copy(zero_v, acc_sh.at[pl.ds(sid * RPT + w * 16, 16)])
    pltpu.sync_copy(czero_v, cnt_sh.at[pl.ds(sid * RPT, RPT)])
    plsc.subcore_barrier()

    for g in range(NG):
        row0 = sid * NCH + g * GCH
        base = pl.multiple_of((row0 // 8) * 8, 8)
        off = row0 - base
        pltpu.sync_copy(src_hbm.at[pl.ds(pl.multiple_of(cid * SLAB + base, 8),
                                         IDXR)], src_v)
        pltpu.sync_copy(dst_hbm.at[pl.ds(base, IDXR)], dst_v)

        def chunk_body(j, carry):
            pltpu.sync_copy(x2_hbm.at[src_v.at[off + j]], rows_v)
            pltpu.sync_copy(rows_v, acc_sh.at[dst_v.at[off + j]], add=True)
            pltpu.sync_copy(ones_v, cnt_sh.at[dst_v.at[off + j]], add=True)
            return carry

        lax.fori_loop(0, GCH, chunk_body, 0)
    plsc.subcore_barrier()

    for w in range(NW):
        r0 = sid * RPT + w * WCH
        pltpu.sync_copy(acc_sh.at[pl.ds(r0, WCH)], stage_v)
        pltpu.sync_copy(stage_v, agg_hbm.at[cid, pl.ds(r0, WCH)])

    @pl.when(cid == 0)
    def _():
        pltpu.sync_copy(cnt_sh.at[pl.ds(sid * RPT, RPT)], cstage_v)
        pltpu.sync_copy(cstage_v, cnt_hbm.at[pl.ds(sid * RPT, RPT)])


def _sc_scatter(x2, src_rows, dst_rows):
    mesh = plsc.VectorSubcoreMesh(core_axis_name="c", subcore_axis_name="s")
    return pl.kernel(
        _sc_scatter_kernel,
        out_type=[
            jax.ShapeDtypeStruct((NC, N_PAD, DH), jnp.float32),
            jax.ShapeDtypeStruct((N_PAD,), jnp.float32),
        ],
        mesh=mesh,
        scratch_types=[
            pltpu.VMEM((IDXR, CH), jnp.int32),
            pltpu.VMEM((IDXR, CH), jnp.int32),
            pltpu.VMEM((CH, DH), jnp.float32),
            pltpu.VMEM((CH,), jnp.float32),
            pltpu.VMEM((16, DH), jnp.float32),
            pltpu.VMEM((RPT,), jnp.float32),
            pltpu.VMEM((WCH, DH), jnp.float32),
            pltpu.VMEM((RPT,), jnp.float32),
            pltpu.VMEM_SHARED((N_PAD, DH), jnp.float32),
            pltpu.VMEM_SHARED((N_PAD,), jnp.float32),
        ],
    )(x2, src_rows, dst_rows)


BR = 1000


def _tc_combine_kernel(x_ref, al_ref, ah_ref, cnt_ref, wl_ref, wr_ref, b_ref,
                       o_ref):
    cnt = jnp.maximum(cnt_ref[...], 1.0)
    mean = jnp.concatenate([al_ref[...], ah_ref[...]], axis=1) / cnt
    h = lax.dot_general(mean, wl_ref[...],
                        (((1,), (1,)), ((), ())),
                        preferred_element_type=jnp.float32)
    h = h + lax.dot_general(x_ref[...], wr_ref[...],
                            (((1,), (1,)), ((), ())),
                            preferred_element_type=jnp.float32)
    h = h + b_ref[...]
    h = jnp.where(h > 0, h, 0.01 * h)
    o_ref[...] = h + x_ref[...]


def _tc_combine(x, agg_lo, agg_hi, cnt, W_l, b_l, W_r):
    grid = (N // BR,)
    return pl.pallas_call(
        _tc_combine_kernel,
        out_shape=jax.ShapeDtypeStruct((N, H), jnp.float32),
        grid=grid,
        in_specs=[
            pl.BlockSpec((BR, D), lambda i: (i, 0)),
            pl.BlockSpec((BR, DH), lambda i: (i, 0)),
            pl.BlockSpec((BR, DH), lambda i: (i, 0)),
            pl.BlockSpec((BR, 1), lambda i: (i, 0)),
            pl.BlockSpec((H, D), lambda i: (0, 0)),
            pl.BlockSpec((H, D), lambda i: (0, 0)),
            pl.BlockSpec((1, H), lambda i: (0, 0)),
        ],
        out_specs=pl.BlockSpec((BR, H), lambda i: (i, 0)),
    )(x, agg_lo, agg_hi, cnt, W_l, W_r, b_l.reshape(1, H))


def kernel(x, edge_index, W_l, b_l, W_r):
    src = edge_index[0]
    dst = edge_index[1]

    x2 = x.reshape(N, 2, DH).transpose(1, 0, 2).reshape(2 * N, DH)
    src_c = src.reshape(NROWCH, CH)
    src_rows = jnp.concatenate([src_c, src_c + N], axis=0)
    dst_rows = dst.reshape(NROWCH, CH)

    agg, cnt = _sc_scatter(x2, src_rows, dst_rows)

    agg_lo = agg[0, :N]
    agg_hi = agg[1, :N]
    cnt2 = cnt[:N].reshape(N, 1)
    return _tc_combine(x, agg_lo, agg_hi, cnt2, W_l, b_l, W_r)


# speedup vs baseline: 6.5849x; 1.2123x over previous
import functools

import jax
import jax.numpy as jnp
from jax import lax
from jax.experimental import pallas as pl
from jax.experimental.pallas import tpu as pltpu
from jax.experimental.pallas import tpu_sc as plsc

N = 10000
E = 160000
D = 256
H = 256

NC = 2
NS = 16
DH = D // 2
CH = 80
EPT = E // NS
NCH = EPT // CH
NROWCH = E // CH
N_PAD = 10240
GCH = 25
NG = NCH // GCH
IDXR = 32
SLAB = NROWCH
RPT = N_PAD // NS
WCH = 64
NW = RPT // WCH


def _sc_scatter_kernel(x2_hbm, src_hbm, dst_hbm, agg_hbm, cnt_hbm,
                       src_v, dst_v, rows_v, ones_v, stage_v, cstage_v,
                       gsem, acc_sh, cnt_sh):
    cid = lax.axis_index("c")
    sid = lax.axis_index("s")

    zf = jnp.zeros((16,), jnp.float32)
    for r in range(16):
        for c in range(DH // 16):
            stage_v[r, pl.ds(c * 16, 16)] = zf
    for c in range(RPT // 16):
        cstage_v[pl.ds(c * 16, 16)] = zf
    of = jnp.ones((16,), jnp.float32)
    for c in range(CH // 16):
        ones_v[pl.ds(c * 16, 16)] = of

    for w in range(RPT // 16):
        pltpu.sync_copy(stage_v, acc_sh.at[pl.ds(sid * RPT + w * 16, 16)])
    pltpu.sync_copy(cstage_v, cnt_sh.at[pl.ds(sid * RPT, RPT)])
    plsc.subcore_barrier()

    for g in range(NG):
        row0 = sid * NCH + g * GCH
        base = pl.multiple_of((row0 // 8) * 8, 8)
        off = row0 - base
        pltpu.sync_copy(src_hbm.at[pl.ds(pl.multiple_of(cid * SLAB + base, 8),
                                         IDXR)], src_v)
        pltpu.sync_copy(dst_hbm.at[pl.ds(base, IDXR)], dst_v)

        pltpu.async_copy(x2_hbm.at[src_v.at[off]], rows_v.at[0], gsem)

        def chunk_body(j, carry):
            b = j % 2
            pltpu.make_async_copy(x2_hbm.at[src_v.at[off + j]],
                                  rows_v.at[b], gsem).wait()

            @pl.when(j + 1 < GCH)
            def _():
                pltpu.async_copy(x2_hbm.at[src_v.at[off + j + 1]],
                                 rows_v.at[1 - b], gsem)

            pltpu.sync_copy(rows_v.at[b], acc_sh.at[dst_v.at[off + j]],
                            add=True)
            pltpu.sync_copy(ones_v, cnt_sh.at[dst_v.at[off + j]], add=True)
            return carry

        lax.fori_loop(0, GCH, chunk_body, 0)
    plsc.subcore_barrier()

    for w in range(RPT // 16):
        r0 = sid * RPT + w * 16
        pltpu.sync_copy(acc_sh.at[pl.ds(r0, 16)], stage_v)
        pltpu.sync_copy(stage_v, agg_hbm.at[cid, pl.ds(r0, 16)])

    @pl.when(cid == 0)
    def _():
        pltpu.sync_copy(cnt_sh.at[pl.ds(sid * RPT, RPT)], cstage_v)
        pltpu.sync_copy(cstage_v, cnt_hbm.at[pl.ds(sid * RPT, RPT)])


def _sc_scatter(x2, src_rows, dst_rows):
    mesh = plsc.VectorSubcoreMesh(core_axis_name="c", subcore_axis_name="s")
    return pl.kernel(
        _sc_scatter_kernel,
        out_type=[
            jax.ShapeDtypeStruct((NC, N_PAD, DH), jnp.float32),
            jax.ShapeDtypeStruct((N_PAD,), jnp.float32),
        ],
        mesh=mesh,
        scratch_types=[
            pltpu.VMEM((IDXR, CH), jnp.int32),
            pltpu.VMEM((IDXR, CH), jnp.int32),
            pltpu.VMEM((2, CH, DH), jnp.float32),
            pltpu.VMEM((CH,), jnp.float32),
            pltpu.VMEM((16, DH), jnp.float32),
            pltpu.VMEM((RPT,), jnp.float32),
            pltpu.SemaphoreType.DMA,
            pltpu.VMEM_SHARED((N_PAD, DH), jnp.float32),
            pltpu.VMEM_SHARED((N_PAD,), jnp.float32),
        ],
    )(x2, src_rows, dst_rows)


BR = 1000


def _tc_combine_kernel(x_ref, al_ref, ah_ref, cnt_ref, wl_ref, wr_ref, b_ref,
                       o_ref):
    cnt = jnp.maximum(cnt_ref[...], 1.0)
    mean = jnp.concatenate([al_ref[...], ah_ref[...]], axis=1) / cnt
    h = lax.dot_general(mean, wl_ref[...],
                        (((1,), (1,)), ((), ())),
                        preferred_element_type=jnp.float32)
    h = h + lax.dot_general(x_ref[...], wr_ref[...],
                            (((1,), (1,)), ((), ())),
                            preferred_element_type=jnp.float32)
    h = h + b_ref[...]
    h = jnp.where(h > 0, h, 0.01 * h)
    o_ref[...] = h + x_ref[...]


def _tc_combine(x, agg_lo, agg_hi, cnt, W_l, b_l, W_r):
    grid = (N // BR,)
    return pl.pallas_call(
        _tc_combine_kernel,
        out_shape=jax.ShapeDtypeStruct((N, H), jnp.float32),
        grid=grid,
        in_specs=[
            pl.BlockSpec((BR, D), lambda i: (i, 0)),
            pl.BlockSpec((BR, DH), lambda i: (i, 0)),
            pl.BlockSpec((BR, DH), lambda i: (i, 0)),
            pl.BlockSpec((BR, 1), lambda i: (i, 0)),
            pl.BlockSpec((H, D), lambda i: (0, 0)),
            pl.BlockSpec((H, D), lambda i: (0, 0)),
            pl.BlockSpec((1, H), lambda i: (0, 0)),
        ],
        out_specs=pl.BlockSpec((BR, H), lambda i: (i, 0)),
    )(x, agg_lo, agg_hi, cnt, W_l, W_r, b_l.reshape(1, H))


def kernel(x, edge_index, W_l, b_l, W_r):
    src = edge_index[0]
    dst = edge_index[1]

    x2 = x.reshape(2 * N, DH)
    src_c = (src * 2).reshape(NROWCH, CH)
    src_rows = jnp.concatenate([src_c, src_c + 1], axis=0)
    dst_rows = dst.reshape(NROWCH, CH)

    agg, cnt = _sc_scatter(x2, src_rows, dst_rows)

    agg_lo = agg[0, :N]
    agg_hi = agg[1, :N]
    cnt2 = cnt[:N].reshape(N, 1)
    return _tc_combine(x, agg_lo, agg_hi, cnt2, W_l, b_l, W_r)


# speedup vs baseline: 6.9710x; 1.0586x over previous
import functools

import jax
import jax.numpy as jnp
from jax import lax
from jax.experimental import pallas as pl
from jax.experimental.pallas import tpu as pltpu
from jax.experimental.pallas import tpu_sc as plsc

N = 10000
E = 160000
D = 256
H = 256

NC = 2
NS = 16
DH = D // 2
CH = 80
EPT = E // NS
NCH = EPT // CH
NROWCH = E // CH
N_PAD = 10240
GCH = 25
NG = NCH // GCH
IDXR = 32
SLAB = NROWCH
RPT = N_PAD // NS
WCH = 64
NW = RPT // WCH


def _sc_scatter_kernel(x2_hbm, src_hbm, dst_hbm, agg_hbm, cnt_hbm,
                       src_v, dst_v, rows_v, ones_v, stage_v, cstage_v,
                       gsem, isem, csem, acc_sh, cnt_sh):
    cid = lax.axis_index("c")
    sid = lax.axis_index("s")

    zf = jnp.zeros((16,), jnp.float32)
    for r in range(16):
        for c in range(DH // 16):
            stage_v[r, pl.ds(c * 16, 16)] = zf
    for c in range(RPT // 16):
        cstage_v[pl.ds(c * 16, 16)] = zf
    of = jnp.ones((16,), jnp.float32)
    for c in range(CH // 16):
        ones_v[pl.ds(c * 16, 16)] = of

    for w in range(RPT // 16):
        pltpu.sync_copy(stage_v, acc_sh.at[pl.ds(sid * RPT + w * 16, 16)])
    pltpu.sync_copy(cstage_v, cnt_sh.at[pl.ds(sid * RPT, RPT)])
    plsc.subcore_barrier()

    def idx_slices(g):
        row0 = sid * NCH + g * GCH
        base = pl.multiple_of((row0 // 8) * 8, 8)
        off = row0 - base
        src_sl = src_hbm.at[pl.ds(pl.multiple_of(cid * SLAB + base, 8), IDXR)]
        dst_sl = dst_hbm.at[pl.ds(base, IDXR)]
        return src_sl, dst_sl, off

    def stage(g, b):
        src_sl, dst_sl, _ = idx_slices(g)
        pltpu.async_copy(src_sl, src_v.at[b], isem)
        pltpu.async_copy(dst_sl, dst_v.at[b], isem)

    stage(0, 0)
    for g in range(NG):
        ib = g % 2
        src_sl, dst_sl, off = idx_slices(g)
        pltpu.make_async_copy(src_sl, src_v.at[ib], isem).wait()
        pltpu.make_async_copy(dst_sl, dst_v.at[ib], isem).wait()
        if g + 1 < NG:
            stage(g + 1, 1 - ib)

        pltpu.async_copy(x2_hbm.at[src_v.at[ib, off]], rows_v.at[0], gsem)

        def chunk_body(j, carry):
            b = j % 2
            pltpu.make_async_copy(x2_hbm.at[src_v.at[ib, off + j]],
                                  rows_v.at[b], gsem).wait()

            @pl.when(j + 1 < GCH)
            def _():
                pltpu.async_copy(x2_hbm.at[src_v.at[ib, off + j + 1]],
                                 rows_v.at[1 - b], gsem)

            pltpu.sync_copy(ones_v, cnt_sh.at[dst_v.at[ib, off + j]],
                            add=True)
            pltpu.sync_copy(rows_v.at[b], acc_sh.at[dst_v.at[ib, off + j]],
                            add=True)
            return carry

        lax.fori_loop(0, GCH, chunk_body, 0)
    plsc.subcore_barrier()

    for w in range(RPT // 16):
        r0 = sid * RPT + w * 16
        pltpu.sync_copy(acc_sh.at[pl.ds(r0, 16)], stage_v)
        pltpu.sync_copy(stage_v, agg_hbm.at[cid, pl.ds(r0, 16)])

    @pl.when(cid == 0)
    def _():
        pltpu.sync_copy(cnt_sh.at[pl.ds(sid * RPT, RPT)], cstage_v)
        pltpu.sync_copy(cstage_v, cnt_hbm.at[pl.ds(sid * RPT, RPT)])


def _sc_scatter(x2, src_rows, dst_rows):
    mesh = plsc.VectorSubcoreMesh(core_axis_name="c", subcore_axis_name="s")
    return pl.kernel(
        _sc_scatter_kernel,
        out_type=[
            jax.ShapeDtypeStruct((NC, N_PAD, DH), jnp.float32),
            jax.ShapeDtypeStruct((N_PAD,), jnp.float32),
        ],
        mesh=mesh,
        scratch_types=[
            pltpu.VMEM((2, IDXR, CH), jnp.int32),
            pltpu.VMEM((2, IDXR, CH), jnp.int32),
            pltpu.VMEM((2, CH, DH), jnp.float32),
            pltpu.VMEM((CH,), jnp.float32),
            pltpu.VMEM((16, DH), jnp.float32),
            pltpu.VMEM((RPT,), jnp.float32),
            pltpu.SemaphoreType.DMA,
            pltpu.SemaphoreType.DMA,
            pltpu.SemaphoreType.DMA,
            pltpu.VMEM_SHARED((N_PAD, DH), jnp.float32),
            pltpu.VMEM_SHARED((N_PAD,), jnp.float32),
        ],
    )(x2, src_rows, dst_rows)


BR = 1000


def _tc_combine_kernel(x_ref, al_ref, ah_ref, cnt_ref, wl_ref, wr_ref, b_ref,
                       o_ref):
    cnt = jnp.maximum(cnt_ref[...], 1.0)
    mean = jnp.concatenate([al_ref[0], ah_ref[0]], axis=1) / cnt
    h = lax.dot_general(mean, wl_ref[...],
                        (((1,), (1,)), ((), ())),
                        preferred_element_type=jnp.float32)
    h = h + lax.dot_general(x_ref[...], wr_ref[...],
                            (((1,), (1,)), ((), ())),
                            preferred_element_type=jnp.float32)
    h = h + b_ref[...]
    h = jnp.where(h > 0, h, 0.01 * h)
    o_ref[...] = h + x_ref[...]


def _tc_combine(x, agg, cnt, W_l, b_l, W_r):
    grid = (N // BR,)
    return pl.pallas_call(
        _tc_combine_kernel,
        out_shape=jax.ShapeDtypeStruct((N, H), jnp.float32),
        grid=grid,
        in_specs=[
            pl.BlockSpec((BR, D), lambda i: (i, 0)),
            pl.BlockSpec((1, BR, DH), lambda i: (0, i, 0)),
            pl.BlockSpec((1, BR, DH), lambda i: (1, i, 0)),
            pl.BlockSpec((BR, 1), lambda i: (i, 0)),
            pl.BlockSpec((H, D), lambda i: (0, 0)),
            pl.BlockSpec((H, D), lambda i: (0, 0)),
            pl.BlockSpec((1, H), lambda i: (0, 0)),
        ],
        out_specs=pl.BlockSpec((BR, H), lambda i: (i, 0)),
    )(x, agg, agg, cnt, W_l, W_r, b_l.reshape(1, H))


def kernel(x, edge_index, W_l, b_l, W_r):
    src = edge_index[0]
    dst = edge_index[1]

    x2 = x.reshape(2 * N, DH)
    src_c = (src * 2).reshape(NROWCH, CH)
    src_rows = jnp.concatenate([src_c, src_c + 1], axis=0)
    dst_rows = dst.reshape(NROWCH, CH)

    agg, cnt = _sc_scatter(x2, src_rows, dst_rows)
    return _tc_combine(x, agg, cnt.reshape(N_PAD, 1), W_l, b_l, W_r)


# speedup vs baseline: 6.9727x; 1.0002x over previous
import functools

import jax
import jax.numpy as jnp
from jax import lax
from jax.experimental import pallas as pl
from jax.experimental.pallas import tpu as pltpu
from jax.experimental.pallas import tpu_sc as plsc

N = 10000
E = 160000
D = 256
H = 256

NC = 2
NS = 16
DH = D // 2
CH = 80
EPT = E // NS
NCH = EPT // CH
NROWCH = E // CH
N_PAD = 10240
GCH = 25
CNT_SPLIT = 63
NG = NCH // GCH
IDXR = 32
SLAB = NROWCH
RPT = N_PAD // NS
WCH = 64
NW = RPT // WCH


def _sc_scatter_kernel(x2_hbm, src_hbm, dst_hbm, agg_hbm, cnt_hbm,
                       src_v, dst_v, rows_v, ones_v, stage_v, cstage_v,
                       gsem, isem, ssem, zsem, acc_sh, cnt_sh):
    cid = lax.axis_index("c")
    sid = lax.axis_index("s")

    zf = jnp.zeros((16,), jnp.float32)
    for r in range(16):
        for c in range(DH // 16):
            stage_v[r, pl.ds(c * 16, 16)] = zf
    for c in range(RPT // 16):
        cstage_v[pl.ds(c * 16, 16)] = zf
    of = jnp.ones((16,), jnp.float32)
    for c in range(CH // 16):
        ones_v[pl.ds(c * 16, 16)] = of

    def idx_slices(g):
        row0 = sid * NCH + g * GCH
        base = pl.multiple_of((row0 // 8) * 8, 8)
        off = row0 - base
        src_sl = src_hbm.at[pl.ds(pl.multiple_of(cid * SLAB + base, 8), IDXR)]
        dst_sl = dst_hbm.at[pl.ds(base, IDXR)]
        return src_sl, dst_sl, off

    def stage(g, b):
        src_sl, dst_sl, _ = idx_slices(g)
        pltpu.async_copy(src_sl, src_v.at[b], isem)
        pltpu.async_copy(dst_sl, dst_v.at[b], isem)

    stage(0, 0)

    for wb in range(RPT // 16 // 5):
        for w in range(5):
            r0 = sid * RPT + (wb * 5 + w) * 16
            pltpu.async_copy(stage_v, acc_sh.at[pl.ds(r0, 16)], zsem)
        for w in range(5):
            r0 = sid * RPT + (wb * 5 + w) * 16
            pltpu.make_async_copy(stage_v, acc_sh.at[pl.ds(r0, 16)],
                                  zsem).wait()
    pltpu.sync_copy(cstage_v, cnt_sh.at[pl.ds(sid * RPT, RPT)])
    plsc.subcore_barrier()

    for g in range(NG):
        ib = g % 2
        src_sl, dst_sl, off = idx_slices(g)
        pltpu.make_async_copy(src_sl, src_v.at[ib], isem).wait()
        pltpu.make_async_copy(dst_sl, dst_v.at[ib], isem).wait()
        if g + 1 < NG:
            stage(g + 1, 1 - ib)

        pltpu.async_copy(x2_hbm.at[src_v.at[ib, off]], rows_v.at[0], gsem)

        def chunk_body(j, carry):
            b = j % 2
            pltpu.make_async_copy(x2_hbm.at[src_v.at[ib, off + j]],
                                  rows_v.at[b], gsem).wait()

            if g * GCH + GCH <= CNT_SPLIT:
                do_cnt = cid == 0
            elif g * GCH >= CNT_SPLIT:
                do_cnt = cid == 1
            else:
                do_cnt = (cid == 0) == (j < CNT_SPLIT - g * GCH)

            @pl.when(do_cnt)
            def _():
                pltpu.sync_copy(ones_v, cnt_sh.at[dst_v.at[ib, off + j]],
                                add=True)

            pltpu.async_copy(rows_v.at[b], acc_sh.at[dst_v.at[ib, off + j]],
                             ssem, add=True)

            @pl.when(j >= 1)
            def _():
                pltpu.make_async_copy(
                    rows_v.at[1 - b], acc_sh.at[dst_v.at[ib, off + j - 1]],
                    ssem).wait()

            @pl.when(j + 1 < GCH)
            def _():
                pltpu.async_copy(x2_hbm.at[src_v.at[ib, off + j + 1]],
                                 rows_v.at[1 - b], gsem)

            return carry

        lax.fori_loop(0, GCH, chunk_body, 0)
        pltpu.make_async_copy(rows_v.at[(GCH - 1) % 2],
                              acc_sh.at[dst_v.at[ib, off + GCH - 1]],
                              ssem).wait()
    plsc.subcore_barrier()

    pltpu.sync_copy(acc_sh.at[pl.ds(sid * RPT, RPT)],
                    agg_hbm.at[cid, pl.ds(sid * RPT, RPT)])
    pltpu.sync_copy(cnt_sh.at[pl.ds(sid * RPT, RPT)],
                    cnt_hbm.at[cid, pl.ds(sid * RPT, RPT)])


def _sc_scatter(x2, src_rows, dst_rows):
    mesh = plsc.VectorSubcoreMesh(core_axis_name="c", subcore_axis_name="s")
    return pl.kernel(
        _sc_scatter_kernel,
        out_type=[
            jax.ShapeDtypeStruct((NC, N_PAD, DH), jnp.float32),
            jax.ShapeDtypeStruct((NC, N_PAD), jnp.float32),
        ],
        mesh=mesh,
        scratch_types=[
            pltpu.VMEM((2, IDXR, CH), jnp.int32),
            pltpu.VMEM((2, IDXR, CH), jnp.int32),
            pltpu.VMEM((2, CH, DH), jnp.float32),
            pltpu.VMEM((CH,), jnp.float32),
            pltpu.VMEM((16, DH), jnp.float32),
            pltpu.VMEM((RPT,), jnp.float32),
            pltpu.SemaphoreType.DMA,
            pltpu.SemaphoreType.DMA,
            pltpu.SemaphoreType.DMA,
            pltpu.SemaphoreType.DMA,
            pltpu.VMEM_SHARED((N_PAD, DH), jnp.float32),
            pltpu.VMEM_SHARED((N_PAD,), jnp.float32),
        ],
    )(x2, src_rows, dst_rows)


BR = 1000


def _tc_combine_kernel(x_ref, al_ref, ah_ref, c0_ref, c1_ref, wl_ref, wr_ref,
                       b_ref, o_ref):
    cnt = jnp.maximum(c0_ref[0] + c1_ref[0], 1.0)
    mean = jnp.concatenate([al_ref[0], ah_ref[0]], axis=1) / cnt
    h = lax.dot_general(mean, wl_ref[...],
                        (((1,), (1,)), ((), ())),
                        preferred_element_type=jnp.float32)
    h = h + lax.dot_general(x_ref[...], wr_ref[...],
                            (((1,), (1,)), ((), ())),
                            preferred_element_type=jnp.float32)
    h = h + b_ref[...]
    h = jnp.where(h > 0, h, 0.01 * h)
    o_ref[...] = h + x_ref[...]


def _tc_combine(x, agg, cnt, W_l, b_l, W_r):
    grid = (N // BR,)
    return pl.pallas_call(
        _tc_combine_kernel,
        out_shape=jax.ShapeDtypeStruct((N, H), jnp.float32),
        grid=grid,
        in_specs=[
            pl.BlockSpec((BR, D), lambda i: (i, 0)),
            pl.BlockSpec((1, BR, DH), lambda i: (0, i, 0)),
            pl.BlockSpec((1, BR, DH), lambda i: (1, i, 0)),
            pl.BlockSpec((1, BR, 1), lambda i: (0, i, 0)),
            pl.BlockSpec((1, BR, 1), lambda i: (1, i, 0)),
            pl.BlockSpec((H, D), lambda i: (0, 0)),
            pl.BlockSpec((H, D), lambda i: (0, 0)),
            pl.BlockSpec((1, H), lambda i: (0, 0)),
        ],
        out_specs=pl.BlockSpec((BR, H), lambda i: (i, 0)),
    )(x, agg, agg, cnt, cnt, W_l, W_r, b_l.reshape(1, H))


def kernel(x, edge_index, W_l, b_l, W_r):
    src = edge_index[0]
    dst = edge_index[1]

    x2 = x.reshape(2 * N, DH)
    src_c = (src * 2).reshape(NROWCH, CH)
    src_rows = jnp.concatenate([src_c, src_c + 1], axis=0)
    dst_rows = dst.reshape(NROWCH, CH)

    agg, cnt = _sc_scatter(x2, src_rows, dst_rows)
    return _tc_combine(x, agg, cnt.reshape(NC, N_PAD, 1), W_l, b_l, W_r)


# speedup vs baseline: 7.8883x; 1.1313x over previous
import functools

import jax
import jax.numpy as jnp
from jax import lax
from jax.experimental import pallas as pl
from jax.experimental.pallas import tpu as pltpu
from jax.experimental.pallas import tpu_sc as plsc

N = 10000
E = 160000
D = 256
H = 256

NC = 2
NS = 16
DH = D // 2
CH = 128
E_PAD = 163840
EPT = E_PAD // NS
NCH = EPT // CH
NROWCH = E_PAD // CH
N_PAD = 10240
GCH = 16
NG = NCH // GCH
CNT_SPLIT = 40
RPT = N_PAD // NS


def _sc_scatter_kernel(x2_hbm, src_hbm, dst_hbm, agg_hbm, cnt_hbm,
                       src_v, dst_v, rows_v, ones_v, stage_v, cstage_v,
                       gsem, isem, ssem, zsem, acc_sh, cnt_sh):
    cid = lax.axis_index("c")
    sid = lax.axis_index("s")

    zf = jnp.zeros((16,), jnp.float32)
    for r in range(16):
        for c in range(DH // 16):
            stage_v[r, pl.ds(c * 16, 16)] = zf
    for c in range(RPT // 16):
        cstage_v[pl.ds(c * 16, 16)] = zf
    of = jnp.ones((16,), jnp.float32)
    for c in range(CH // 16):
        ones_v[pl.ds(c * 16, 16)] = of

    def stage(g, b):
        row0 = sid * NCH + g * GCH
        pltpu.async_copy(src_hbm.at[pl.ds(row0, GCH)], src_v.at[b], isem)
        pltpu.async_copy(dst_hbm.at[pl.ds(row0, GCH)], dst_v.at[b], isem)

    stage(0, 0)

    for wb in range(RPT // 16 // 5):
        for w in range(5):
            r0 = sid * RPT + (wb * 5 + w) * 16
            pltpu.async_copy(stage_v, acc_sh.at[pl.ds(r0, 16)], zsem)
        for w in range(5):
            r0 = sid * RPT + (wb * 5 + w) * 16
            pltpu.make_async_copy(stage_v, acc_sh.at[pl.ds(r0, 16)],
                                  zsem).wait()
    pltpu.sync_copy(cstage_v, cnt_sh.at[pl.ds(sid * RPT, RPT)])
    plsc.subcore_barrier()

    for g in range(NG):
        ib = g % 2
        row0 = sid * NCH + g * GCH
        pltpu.make_async_copy(src_hbm.at[pl.ds(row0, GCH)], src_v.at[ib],
                              isem).wait()
        pltpu.make_async_copy(dst_hbm.at[pl.ds(row0, GCH)], dst_v.at[ib],
                              isem).wait()
        if g + 1 < NG:
            stage(g + 1, 1 - ib)

        def tbody(i, carry):
            r = i // (CH // 16)
            c = (i % (CH // 16)) * 16
            v = src_v[ib, r, pl.ds(c, 16)]
            src_v[ib, r, pl.ds(c, 16)] = v * 2 + cid
            return carry

        lax.fori_loop(0, GCH * (CH // 16), tbody, 0)

        pltpu.async_copy(x2_hbm.at[src_v.at[ib, 0]], rows_v.at[0], gsem)

        def chunk_body(j, carry):
            b = j % 2
            pltpu.make_async_copy(x2_hbm.at[src_v.at[ib, j]],
                                  rows_v.at[b], gsem).wait()

            if (g + 1) * GCH <= CNT_SPLIT:
                do_cnt = cid == 0
            elif g * GCH >= CNT_SPLIT:
                do_cnt = cid == 1
            else:
                do_cnt = (cid == 0) == (j < CNT_SPLIT - g * GCH)

            @pl.when(do_cnt)
            def _():
                pltpu.sync_copy(ones_v, cnt_sh.at[dst_v.at[ib, j]],
                                add=True)

            pltpu.async_copy(rows_v.at[b], acc_sh.at[dst_v.at[ib, j]],
                             ssem, add=True)

            @pl.when(j >= 1)
            def _():
                pltpu.make_async_copy(
                    rows_v.at[1 - b], acc_sh.at[dst_v.at[ib, j - 1]],
                    ssem).wait()

            @pl.when(j + 1 < GCH)
            def _():
                pltpu.async_copy(x2_hbm.at[src_v.at[ib, j + 1]],
                                 rows_v.at[1 - b], gsem)

            return carry

        lax.fori_loop(0, GCH, chunk_body, 0)
        pltpu.make_async_copy(rows_v.at[(GCH - 1) % 2],
                              acc_sh.at[dst_v.at[ib, GCH - 1]],
                              ssem).wait()
    plsc.subcore_barrier()

    pltpu.sync_copy(acc_sh.at[pl.ds(sid * RPT, RPT)],
                    agg_hbm.at[cid, pl.ds(sid * RPT, RPT)])
    pltpu.sync_copy(cnt_sh.at[pl.ds(sid * RPT, RPT)],
                    cnt_hbm.at[cid, pl.ds(sid * RPT, RPT)])


def _sc_scatter(x2, src_rows, dst_rows):
    mesh = plsc.VectorSubcoreMesh(core_axis_name="c", subcore_axis_name="s")
    return pl.kernel(
        _sc_scatter_kernel,
        out_type=[
            jax.ShapeDtypeStruct((NC, N_PAD, DH), jnp.float32),
            jax.ShapeDtypeStruct((NC, N_PAD), jnp.float32),
        ],
        mesh=mesh,
        scratch_types=[
            pltpu.VMEM((2, GCH, CH), jnp.int32),
            pltpu.VMEM((2, GCH, CH), jnp.int32),
            pltpu.VMEM((2, CH, DH), jnp.float32),
            pltpu.VMEM((CH,), jnp.float32),
            pltpu.VMEM((16, DH), jnp.float32),
            pltpu.VMEM((RPT,), jnp.float32),
            pltpu.SemaphoreType.DMA,
            pltpu.SemaphoreType.DMA,
            pltpu.SemaphoreType.DMA,
            pltpu.SemaphoreType.DMA,
            pltpu.VMEM_SHARED((N_PAD, DH), jnp.float32),
            pltpu.VMEM_SHARED((N_PAD,), jnp.float32),
        ],
    )(x2, src_rows, dst_rows)


BR = 1000


def _tc_combine_kernel(x_ref, al_ref, ah_ref, c0_ref, c1_ref, wl_ref, wr_ref,
                       b_ref, o_ref):
    cnt = jnp.maximum(c0_ref[0] + c1_ref[0], 1.0)
    mean = jnp.concatenate([al_ref[0], ah_ref[0]], axis=1) / cnt
    h = lax.dot_general(mean, wl_ref[...],
                        (((1,), (1,)), ((), ())),
                        preferred_element_type=jnp.float32)
    h = h + lax.dot_general(x_ref[...], wr_ref[...],
                            (((1,), (1,)), ((), ())),
                            preferred_element_type=jnp.float32)
    h = h + b_ref[...]
    h = jnp.where(h > 0, h, 0.01 * h)
    o_ref[...] = h + x_ref[...]


def _tc_combine(x, agg, cnt, W_l, b_l, W_r):
    grid = (N // BR,)
    return pl.pallas_call(
        _tc_combine_kernel,
        out_shape=jax.ShapeDtypeStruct((N, H), jnp.float32),
        grid=grid,
        in_specs=[
            pl.BlockSpec((BR, D), lambda i: (i, 0)),
            pl.BlockSpec((1, BR, DH), lambda i: (0, i, 0)),
            pl.BlockSpec((1, BR, DH), lambda i: (1, i, 0)),
            pl.BlockSpec((1, BR, 1), lambda i: (0, i, 0)),
            pl.BlockSpec((1, BR, 1), lambda i: (1, i, 0)),
            pl.BlockSpec((H, D), lambda i: (0, 0)),
            pl.BlockSpec((H, D), lambda i: (0, 0)),
            pl.BlockSpec((1, H), lambda i: (0, 0)),
        ],
        out_specs=pl.BlockSpec((BR, H), lambda i: (i, 0)),
    )(x, agg, agg, cnt, cnt, W_l, W_r, b_l.reshape(1, H))


def kernel(x, edge_index, W_l, b_l, W_r):
    src = edge_index[0]
    dst = edge_index[1]

    x2 = x.reshape(2 * N, DH)

    npad = E_PAD - E
    ar = jnp.arange(npad, dtype=jnp.int32)
    src_rows = jnp.concatenate([src, (ar * 41) % N]).reshape(NROWCH, CH)
    dst_rows = jnp.concatenate([dst, N + (ar % (N_PAD - N))]).reshape(
        NROWCH, CH)

    agg, cnt = _sc_scatter(x2, src_rows, dst_rows)
    return _tc_combine(x, agg, cnt.reshape(NC, N_PAD, 1), W_l, b_l, W_r)
